# baseline (speedup 1.0000x reference)
import functools

import jax
import jax.numpy as jnp
from jax import lax
from jax.experimental import pallas as pl
from jax.experimental.pallas import tpu as pltpu
from jax.experimental.pallas import tpu_sc as plsc

N_REQS = 16
MAX_N_REQS = 64
N_TOKENS = 32768

NC = 1
NS = 16
L = 16
NW = NC * NS
TOK_PER_W = N_TOKENS // NW
VECS_PER_W = TOK_PER_W // L

_INT_INF = 2**31 - 1


N_CHUNKS = 4
CH_TOK = TOK_PER_W // N_CHUNKS
CH_VECS = CH_TOK // L


def _body(idx_hbm, qsl_hbm, nct_hbm, pos_hbm, pos_out_hbm, seq_out_hbm,
          pos_v, qsl_v, idx_v, nct_v, off_v, seq_v, sem_meta,
          sems_in, sems_out):
    wid = lax.axis_index("s") * NC + lax.axis_index("c")
    base = wid * TOK_PER_W

    qsl_v[pl.ds(16, 16)] = jnp.full((16,), _INT_INF, jnp.int32)
    cps_in = [
        pltpu.async_copy(pos_hbm.at[pl.ds(base + i * CH_TOK, CH_TOK)],
                         pos_v.at[pl.ds(i * CH_TOK, CH_TOK)],
                         sems_in.at[i])
        for i in range(N_CHUNKS)
    ]
    cp_qsl = pltpu.async_copy(qsl_hbm, qsl_v.at[pl.ds(0, N_REQS + 1)],
                              sem_meta)
    cp_idx = pltpu.async_copy(idx_hbm, idx_v, sem_meta)
    cp_nct = pltpu.async_copy(nct_hbm, nct_v, sem_meta)
    cp_qsl.wait()
    cp_idx.wait()
    cp_nct.wait()

    qsl0 = qsl_v[pl.ds(0, N_REQS)]
    qsl1 = qsl_v[pl.ds(1, N_REQS)]
    idx = idx_v[...]
    nct_req = plsc.load_gather(nct_v, [idx])
    off_v[...] = nct_req - qsl0

    @pl.when(wid == 0)
    def _():
        seq_v[pl.ds(0, 16)] = nct_req + (qsl1 - qsl0)
        zeros = jnp.zeros((16,), jnp.int32)
        seq_v[pl.ds(16, 16)] = zeros
        seq_v[pl.ds(32, 16)] = zeros
        seq_v[pl.ds(48, 16)] = zeros
        pltpu.sync_copy(seq_v, seq_out_hbm)

    iota = lax.iota(jnp.int32, L)
    t_base = lax.broadcast_in_dim(base, (L,), ()) + iota
    qsl15 = plsc.load_gather(qsl_v, [jnp.full((L,), 15, jnp.int32)])

    cps_out = []
    for i in range(N_CHUNKS):
        cps_in[i].wait()

        @plsc.parallel_loop(i * CH_VECS, (i + 1) * CH_VECS, 1, unroll=4)
        def token_vec(v):
            t = t_base + lax.broadcast_in_dim(v * L, (L,), ())
            c = jnp.where(qsl15 <= t, 16, 0).astype(jnp.int32)
            for step in (8, 4, 2, 1):
                k = c + step
                probe = plsc.load_gather(qsl_v, [k - 1])
                c = jnp.where(probe <= t, k, c)
            valid = (c >= 1) & (c <= N_REQS)
            req_c = jnp.clip(c - 1, 0, N_REQS - 1)
            off = plsc.load_gather(off_v, [req_c])
            newpos = (t + off).astype(jnp.float32)
            cur = pos_v[pl.ds(v * L, L)]
            pos_v[pl.ds(v * L, L)] = jnp.where(valid, newpos, cur)

        cps_out.append(
            pltpu.async_copy(pos_v.at[pl.ds(i * CH_TOK, CH_TOK)],
                             pos_out_hbm.at[pl.ds(base + i * CH_TOK, CH_TOK)],
                             sems_out.at[i]))
    for cp in cps_out:
        cp.wait()


@jax.jit
def _run(idx_mapping, query_start_loc, num_computed_tokens, pos):
    mesh = plsc.VectorSubcoreMesh(
        core_axis_name="c", subcore_axis_name="s",
        num_cores=NC, num_subcores=NS)
    f = pl.kernel(
        _body,
        out_type=(
            jax.ShapeDtypeStruct((N_TOKENS,), jnp.float32),
            jax.ShapeDtypeStruct((MAX_N_REQS,), jnp.int32),
        ),
        mesh=mesh,
        scratch_types=[
            pltpu.VMEM((TOK_PER_W,), jnp.float32),
            pltpu.VMEM((32,), jnp.int32),
            pltpu.VMEM((N_REQS,), jnp.int32),
            pltpu.VMEM((MAX_N_REQS,), jnp.int32),
            pltpu.VMEM((N_REQS,), jnp.int32),
            pltpu.VMEM((MAX_N_REQS,), jnp.int32),
            pltpu.SemaphoreType.DMA,
            pltpu.SemaphoreType.DMA((N_CHUNKS,)),
            pltpu.SemaphoreType.DMA((N_CHUNKS,)),
        ],
        compiler_params=pltpu.CompilerParams(needs_layout_passes=False),
    )
    return f(idx_mapping, query_start_loc, num_computed_tokens, pos)


def kernel(idx_mapping, query_start_loc, num_computed_tokens, pos, seq_lens):
    return _run(idx_mapping, query_start_loc, num_computed_tokens, pos)


# speedup vs baseline: 1.0335x; 1.0335x over previous
import functools

import jax
import jax.numpy as jnp
from jax import lax
from jax.experimental import pallas as pl
from jax.experimental.pallas import tpu as pltpu
from jax.experimental.pallas import tpu_sc as plsc

N_REQS = 16
MAX_N_REQS = 64
N_TOKENS = 32768

NC = 1
NS = 16
L = 16
NW = NC * NS
TOK_PER_W = N_TOKENS // NW
VECS_PER_W = TOK_PER_W // L

_INT_INF = 2**31 - 1


N_CHUNKS = 2
CH_TOK = TOK_PER_W // N_CHUNKS
CH_VECS = CH_TOK // L


def _body(idx_hbm, qsl_hbm, nct_hbm, pos_hbm, pos_out_hbm, seq_out_hbm,
          pos_v, qsl_v, idx_v, nct_v, off_v, seq_v, sem_meta,
          sems_in, sems_out):
    wid = lax.axis_index("s") * NC + lax.axis_index("c")
    base = wid * TOK_PER_W

    qsl_v[pl.ds(16, 16)] = jnp.full((16,), _INT_INF, jnp.int32)
    cps_in = [
        pltpu.async_copy(pos_hbm.at[pl.ds(base + i * CH_TOK, CH_TOK)],
                         pos_v.at[pl.ds(i * CH_TOK, CH_TOK)],
                         sems_in.at[i])
        for i in range(N_CHUNKS)
    ]
    cp_qsl = pltpu.async_copy(qsl_hbm, qsl_v.at[pl.ds(0, N_REQS + 1)],
                              sem_meta)
    cp_idx = pltpu.async_copy(idx_hbm, idx_v, sem_meta)
    cp_nct = pltpu.async_copy(nct_hbm, nct_v, sem_meta)
    cp_qsl.wait()
    cp_idx.wait()
    cp_nct.wait()

    qsl0 = qsl_v[pl.ds(0, N_REQS)]
    qsl1 = qsl_v[pl.ds(1, N_REQS)]
    idx = idx_v[...]
    nct_req = plsc.load_gather(nct_v, [idx])
    off_v[...] = nct_req - qsl0

    @pl.when(wid == 0)
    def _():
        seq_v[pl.ds(0, 16)] = nct_req + (qsl1 - qsl0)
        zeros = jnp.zeros((16,), jnp.int32)
        seq_v[pl.ds(16, 16)] = zeros
        seq_v[pl.ds(32, 16)] = zeros
        seq_v[pl.ds(48, 16)] = zeros
        pltpu.sync_copy(seq_v, seq_out_hbm)

    iota = lax.iota(jnp.int32, L)
    t_base = lax.broadcast_in_dim(base, (L,), ()) + iota
    qsl15 = plsc.load_gather(qsl_v, [jnp.full((L,), 15, jnp.int32)])

    cps_out = []
    for i in range(N_CHUNKS):
        cps_in[i].wait()

        @plsc.parallel_loop(i * CH_VECS, (i + 1) * CH_VECS, 1, unroll=4)
        def token_vec(v):
            t = t_base + lax.broadcast_in_dim(v * L, (L,), ())
            c = jnp.where(qsl15 <= t, 16, 0).astype(jnp.int32)
            for step in (8, 4, 2, 1):
                k = c + step
                probe = plsc.load_gather(qsl_v, [k - 1])
                c = jnp.where(probe <= t, k, c)
            valid = (c >= 1) & (c <= N_REQS)
            req_c = jnp.clip(c - 1, 0, N_REQS - 1)
            off = plsc.load_gather(off_v, [req_c])
            newpos = (t + off).astype(jnp.float32)
            cur = pos_v[pl.ds(v * L, L)]
            pos_v[pl.ds(v * L, L)] = jnp.where(valid, newpos, cur)

        cps_out.append(
            pltpu.async_copy(pos_v.at[pl.ds(i * CH_TOK, CH_TOK)],
                             pos_out_hbm.at[pl.ds(base + i * CH_TOK, CH_TOK)],
                             sems_out.at[i]))
    for cp in cps_out:
        cp.wait()


@jax.jit
def _run(idx_mapping, query_start_loc, num_computed_tokens, pos):
    mesh = plsc.VectorSubcoreMesh(
        core_axis_name="c", subcore_axis_name="s",
        num_cores=NC, num_subcores=NS)
    f = pl.kernel(
        _body,
        out_type=(
            jax.ShapeDtypeStruct((N_TOKENS,), jnp.float32),
            jax.ShapeDtypeStruct((MAX_N_REQS,), jnp.int32),
        ),
        mesh=mesh,
        scratch_types=[
            pltpu.VMEM((TOK_PER_W,), jnp.float32),
            pltpu.VMEM((32,), jnp.int32),
            pltpu.VMEM((N_REQS,), jnp.int32),
            pltpu.VMEM((MAX_N_REQS,), jnp.int32),
            pltpu.VMEM((N_REQS,), jnp.int32),
            pltpu.VMEM((MAX_N_REQS,), jnp.int32),
            pltpu.SemaphoreType.DMA,
            pltpu.SemaphoreType.DMA((N_CHUNKS,)),
            pltpu.SemaphoreType.DMA((N_CHUNKS,)),
        ],
        compiler_params=pltpu.CompilerParams(needs_layout_passes=False),
    )
    return f(idx_mapping, query_start_loc, num_computed_tokens, pos)


def kernel(idx_mapping, query_start_loc, num_computed_tokens, pos, seq_lens):
    return _run(idx_mapping, query_start_loc, num_computed_tokens, pos)


# speedup vs baseline: 1.0440x; 1.0102x over previous
import functools

import jax
import jax.numpy as jnp
from jax import lax
from jax.experimental import pallas as pl
from jax.experimental.pallas import tpu as pltpu
from jax.experimental.pallas import tpu_sc as plsc

N_REQS = 16
MAX_N_REQS = 64
N_TOKENS = 32768

NC = 1
NS = 16
L = 16
NW = NC * NS
TOK_PER_W = N_TOKENS // NW
VECS_PER_W = TOK_PER_W // L

_INT_INF = 2**31 - 1


def _body(idx_hbm, qsl_hbm, nct_hbm, pos_hbm, pos_out_hbm, seq_out_hbm,
          pos_v, qsl_v, idx_v, nct_v, off_v, seq_v, sem_meta, sem_pos):
    wid = lax.axis_index("s") * NC + lax.axis_index("c")
    base = wid * TOK_PER_W

    qsl_v[pl.ds(16, 16)] = jnp.full((16,), _INT_INF, jnp.int32)
    cp_pos = pltpu.async_copy(pos_hbm.at[pl.ds(base, TOK_PER_W)], pos_v,
                              sem_pos)
    cp_qsl = pltpu.async_copy(qsl_hbm, qsl_v.at[pl.ds(0, N_REQS + 1)],
                              sem_meta)
    cp_idx = pltpu.async_copy(idx_hbm, idx_v, sem_meta)
    cp_nct = pltpu.async_copy(nct_hbm, nct_v, sem_meta)
    cp_qsl.wait()
    cp_idx.wait()
    cp_nct.wait()

    qsl0 = qsl_v[pl.ds(0, N_REQS)]
    qsl1 = qsl_v[pl.ds(1, N_REQS)]
    idx = idx_v[...]
    nct_req = plsc.load_gather(nct_v, [idx])
    off_v[...] = nct_req - qsl0

    @pl.when(wid == 0)
    def _():
        seq_v[pl.ds(0, 16)] = nct_req + (qsl1 - qsl0)
        zeros = jnp.zeros((16,), jnp.int32)
        seq_v[pl.ds(16, 16)] = zeros
        seq_v[pl.ds(32, 16)] = zeros
        seq_v[pl.ds(48, 16)] = zeros
        pltpu.sync_copy(seq_v, seq_out_hbm)

    iota = lax.iota(jnp.int32, L)
    t_base = lax.broadcast_in_dim(base, (L,), ()) + iota
    qsl15 = plsc.load_gather(qsl_v, [jnp.full((L,), 15, jnp.int32)])

    cp_pos.wait()

    @plsc.parallel_loop(0, VECS_PER_W, 1, unroll=8)
    def token_vec(v):
        t = t_base + lax.broadcast_in_dim(v * L, (L,), ())
        c = jnp.where(qsl15 <= t, 16, 0).astype(jnp.int32)
        for step in (8, 4, 2, 1):
            k = c + step
            probe = plsc.load_gather(qsl_v, [k - 1])
            c = jnp.where(probe <= t, k, c)
        valid = (c >= 1) & (c <= N_REQS)
        req_c = jnp.clip(c - 1, 0, N_REQS - 1)
        off = plsc.load_gather(off_v, [req_c])
        newpos = (t + off).astype(jnp.float32)
        cur = pos_v[pl.ds(v * L, L)]
        pos_v[pl.ds(v * L, L)] = jnp.where(valid, newpos, cur)

    pltpu.sync_copy(pos_v, pos_out_hbm.at[pl.ds(base, TOK_PER_W)])


@jax.jit
def _run(idx_mapping, query_start_loc, num_computed_tokens, pos):
    mesh = plsc.VectorSubcoreMesh(
        core_axis_name="c", subcore_axis_name="s",
        num_cores=NC, num_subcores=NS)
    f = pl.kernel(
        _body,
        out_type=(
            jax.ShapeDtypeStruct((N_TOKENS,), jnp.float32),
            jax.ShapeDtypeStruct((MAX_N_REQS,), jnp.int32),
        ),
        mesh=mesh,
        scratch_types=[
            pltpu.VMEM((TOK_PER_W,), jnp.float32),
            pltpu.VMEM((32,), jnp.int32),
            pltpu.VMEM((N_REQS,), jnp.int32),
            pltpu.VMEM((MAX_N_REQS,), jnp.int32),
            pltpu.VMEM((N_REQS,), jnp.int32),
            pltpu.VMEM((MAX_N_REQS,), jnp.int32),
            pltpu.SemaphoreType.DMA,
            pltpu.SemaphoreType.DMA,
        ],
        compiler_params=pltpu.CompilerParams(needs_layout_passes=False),
    )
    return f(idx_mapping, query_start_loc, num_computed_tokens, pos)


def kernel(idx_mapping, query_start_loc, num_computed_tokens, pos, seq_lens):
    return _run(idx_mapping, query_start_loc, num_computed_tokens, pos)
